# Initial kernel scaffold; baseline (speedup 1.0000x reference)
import jax
import jax.numpy as jnp
from jax.experimental import pallas as pl


def kernel(inputs):
    raise NotImplementedError("write your pallas kernel here")


# speedup vs baseline: 124.4361x; 124.4361x over previous
import functools

import jax
import jax.numpy as jnp
from jax import lax
from jax.experimental import pallas as pl
from jax.experimental.pallas import tpu as pltpu
from jax.experimental.pallas import tpu_sc as plsc

NCLS = 21
NB = 16
NPIX = 512 * 512
CH = 32768
NCHUNK = NPIX // CH
VEC = CH // 16

_mesh = plsc.VectorSubcoreMesh(core_axis_name="c", subcore_axis_name="s")


@functools.partial(
    pl.kernel,
    out_type=jax.ShapeDtypeStruct((NB, NPIX), jnp.float32),
    mesh=_mesh,
    compiler_params=pltpu.CompilerParams(needs_layout_passes=False),
    scratch_types=[
        pltpu.VMEM((CH,), jnp.int32),
        pltpu.VMEM((CH,), jnp.float32),
        pltpu.VMEM((512,), jnp.float32),
        pltpu.VMEM((32,), jnp.float32),
    ],
)
def _sc_weight_map(mask_hbm, out_hbm, buf, obuf, hist, wtab):
    c = lax.axis_index("c")
    s = lax.axis_index("s")

    @pl.when(s < 8)
    def _body():
        b = c * 8 + s
        lane = lax.iota(jnp.int32, 16)
        zeros = jnp.zeros((16,), jnp.float32)
        ones = jnp.ones((16,), jnp.float32)

        for k in range(32):
            hist[pl.ds(16 * k, 16)] = zeros

        def chunk1(ci, carry):
            pltpu.sync_copy(mask_hbm.at[b, pl.ds(ci * CH, CH)], buf)

            def vec1(i, carry2):
                base = i * 64
                for u in range(4):
                    v = buf[pl.ds(base + u * 16, 16)]
                    idx = v * 16 + lane
                    plsc.addupdate_scatter(hist, [idx], ones)
                return carry2

            lax.fori_loop(0, VEC // 4, vec1, 0)
            return carry

        lax.fori_loop(0, NCHUNK, chunk1, 0)

        cls0 = lane
        cls1 = lane + 16
        cnt0 = zeros
        cnt1 = zeros
        for l in range(16):
            cnt0 = cnt0 + plsc.load_gather(hist, [cls0 * 16 + l])
            cnt1 = cnt1 + plsc.load_gather(hist, [cls1 * 16 + l])

        inv0 = jnp.where(cnt0 > 0.0, ones / jnp.maximum(cnt0, 1.0), zeros)
        inv1 = jnp.where(cnt1 > 0.0, ones / jnp.maximum(cnt1, 1.0), zeros)
        denom = lax.broadcast(jnp.sum(inv0 + inv1), (16,))
        wtab[pl.ds(0, 16)] = inv0 / denom
        wtab[pl.ds(16, 16)] = inv1 / denom

        def chunk2(ci, carry):
            pltpu.sync_copy(mask_hbm.at[b, pl.ds(ci * CH, CH)], buf)

            def vec2(i, carry2):
                base = i * 64
                for u in range(4):
                    v = buf[pl.ds(base + u * 16, 16)]
                    obuf[pl.ds(base + u * 16, 16)] = plsc.load_gather(wtab, [v])
                return carry2

            lax.fori_loop(0, VEC // 4, vec2, 0)
            pltpu.sync_copy(obuf, out_hbm.at[b, pl.ds(ci * CH, CH)])
            return carry

        lax.fori_loop(0, NCHUNK, chunk2, 0)


def kernel(inputs):
    mask = inputs.astype(jnp.int32).reshape(NB, NPIX)
    out = _sc_weight_map(mask)
    return out.reshape(NB, 512, 512, 1)


# speedup vs baseline: 281.2208x; 2.2600x over previous
import functools

import jax
import jax.numpy as jnp
from jax import lax
from jax.experimental import pallas as pl
from jax.experimental.pallas import tpu as pltpu
from jax.experimental.pallas import tpu_sc as plsc

NCLS = 21
NB = 16
NPIX = 512 * 512
HPIX = NPIX // 2
CH = 16384
NCH = HPIX // CH
HWORDS = 512

_mesh = plsc.VectorSubcoreMesh(core_axis_name="c", subcore_axis_name="s")


@functools.partial(
    pl.kernel,
    out_type=jax.ShapeDtypeStruct((NB, NPIX), jnp.float32),
    mesh=_mesh,
    compiler_params=pltpu.CompilerParams(needs_layout_passes=False),
    scratch_types=[
        pltpu.VMEM((CH,), jnp.int32),
        pltpu.VMEM((CH,), jnp.int32),
        pltpu.VMEM((CH,), jnp.float32),
        pltpu.VMEM((CH,), jnp.float32),
        pltpu.VMEM((HWORDS,), jnp.float32),
        pltpu.VMEM((HWORDS,), jnp.float32),
        pltpu.VMEM((32,), jnp.float32),
        pltpu.VMEM_SHARED((16 * HWORDS,), jnp.float32),
        pltpu.SemaphoreType.DMA,
        pltpu.SemaphoreType.DMA,
        pltpu.SemaphoreType.DMA,
        pltpu.SemaphoreType.DMA,
    ],
)
def _sc_weight_map(mask_hbm, out_hbm, buf0, buf1, obuf0, obuf1,
                   hist, hist2, wtab, shist, semi0, semi1, semo0, semo1):
    c = lax.axis_index("c")
    s = lax.axis_index("s")
    b = c * 8 + s // 2
    base = (s % 2) * HPIX

    lane = lax.iota(jnp.int32, 16)
    zeros = jnp.zeros((16,), jnp.float32)
    ones = jnp.ones((16,), jnp.float32)

    bufs = [buf0, buf1]
    obufs = [obuf0, obuf1]
    semis = [semi0, semi1]
    semos = [semo0, semo1]

    def off(ci):
        return base + ci * CH

    for k in range(32):
        hist[pl.ds(16 * k, 16)] = zeros

    cps = [
        pltpu.async_copy(mask_hbm.at[b, pl.ds(off(0), CH)], buf0, semi0),
        pltpu.async_copy(mask_hbm.at[b, pl.ds(off(1), CH)], buf1, semi1),
    ]
    for ci in range(NCH):
        k = ci % 2
        cps[k].wait()
        buf = bufs[k]

        @plsc.parallel_loop(0, CH, step=16, unroll=8)
        def _h(i):
            v = buf[pl.ds(i, 16)]
            plsc.addupdate_scatter(hist, [v * 16 + lane], ones)

        if ci + 2 < NCH:
            cps[k] = pltpu.async_copy(
                mask_hbm.at[b, pl.ds(off(ci + 2), CH)], bufs[k], semis[k])

    cps = [
        pltpu.async_copy(mask_hbm.at[b, pl.ds(off(0), CH)], buf0, semi0),
        pltpu.async_copy(mask_hbm.at[b, pl.ds(off(1), CH)], buf1, semi1),
    ]

    pltpu.sync_copy(hist, shist.at[pl.ds(s * HWORDS, HWORDS)])
    plsc.subcore_barrier()
    pltpu.sync_copy(shist.at[pl.ds((s ^ 1) * HWORDS, HWORDS)], hist2)

    cls0 = lane * 16
    cls1 = (lane + 16) * 16
    cnt0 = zeros
    cnt1 = zeros
    for l in range(16):
        cnt0 = cnt0 + plsc.load_gather(hist, [cls0 + l])
        cnt0 = cnt0 + plsc.load_gather(hist2, [cls0 + l])
        cnt1 = cnt1 + plsc.load_gather(hist, [cls1 + l])
        cnt1 = cnt1 + plsc.load_gather(hist2, [cls1 + l])

    inv0 = jnp.where(cnt0 > 0.0, ones / jnp.maximum(cnt0, 1.0), zeros)
    inv1 = jnp.where(cnt1 > 0.0, ones / jnp.maximum(cnt1, 1.0), zeros)
    denom = lax.broadcast(jnp.sum(inv0 + inv1), (16,))
    wtab[pl.ds(0, 16)] = inv0 / denom
    wtab[pl.ds(16, 16)] = inv1 / denom

    wrs = [None, None]
    for ci in range(NCH):
        k = ci % 2
        cps[k].wait()
        if wrs[k] is not None:
            wrs[k].wait()
        buf = bufs[k]
        obuf = obufs[k]

        @plsc.parallel_loop(0, CH, step=16, unroll=8)
        def _g(i):
            v = buf[pl.ds(i, 16)]
            obuf[pl.ds(i, 16)] = plsc.load_gather(wtab, [v])

        wrs[k] = pltpu.async_copy(
            obuf, out_hbm.at[b, pl.ds(off(ci), CH)], semos[k])
        if ci + 2 < NCH:
            cps[k] = pltpu.async_copy(
                mask_hbm.at[b, pl.ds(off(ci + 2), CH)], bufs[k], semis[k])

    wrs[0].wait()
    wrs[1].wait()


def kernel(inputs):
    mask = inputs.astype(jnp.int32).reshape(NB, NPIX)
    out = _sc_weight_map(mask)
    return out.reshape(NB, 512, 512, 1)


# speedup vs baseline: 852.6832x; 3.0321x over previous
import functools

import jax
import jax.numpy as jnp
from jax import lax
from jax.experimental import pallas as pl
from jax.experimental.pallas import tpu as pltpu
from jax.experimental.pallas import tpu_sc as plsc

NCLS = 21
NB = 16
NPIX = 512 * 512
ROWS = NB * NPIX // 128
SROWS = NPIX // 128
WROWS = SROWS // 2
CR = 128
NCH = WROWS // CR
HWORDS = 512

_mesh = plsc.VectorSubcoreMesh(core_axis_name="c", subcore_axis_name="s")


@functools.partial(
    pl.kernel,
    out_type=jax.ShapeDtypeStruct((ROWS, 128), jnp.float32),
    mesh=_mesh,
    compiler_params=pltpu.CompilerParams(needs_layout_passes=False),
    scratch_types=[
        pltpu.VMEM((CR, 128), jnp.int32),
        pltpu.VMEM((CR, 128), jnp.int32),
        pltpu.VMEM((CR, 128), jnp.float32),
        pltpu.VMEM((CR, 128), jnp.float32),
        pltpu.VMEM((HWORDS,), jnp.float32),
        pltpu.VMEM((HWORDS,), jnp.float32),
        pltpu.VMEM((32,), jnp.float32),
        pltpu.VMEM_SHARED((16 * HWORDS,), jnp.float32),
        pltpu.SemaphoreType.DMA,
        pltpu.SemaphoreType.DMA,
        pltpu.SemaphoreType.DMA,
        pltpu.SemaphoreType.DMA,
    ],
)
def _sc_weight_map(mask_hbm, out_hbm, buf0, buf1, obuf0, obuf1,
                   hist, hist2, wtab, shist, semi0, semi1, semo0, semo1):
    c = lax.axis_index("c")
    s = lax.axis_index("s")
    b = c * 8 + s // 2
    rbase = b * SROWS + (s % 2) * WROWS

    lane = lax.iota(jnp.int32, 16)
    zeros = jnp.zeros((16,), jnp.float32)
    ones = jnp.ones((16,), jnp.float32)

    bufs = [buf0, buf1]
    obufs = [obuf0, obuf1]
    semis = [semi0, semi1]
    semos = [semo0, semo1]

    def row0(ci):
        return rbase + ci * CR

    for k in range(32):
        hist[pl.ds(16 * k, 16)] = zeros

    cps = [
        pltpu.async_copy(mask_hbm.at[pl.ds(row0(0), CR), :], buf0, semi0),
        pltpu.async_copy(mask_hbm.at[pl.ds(row0(1), CR), :], buf1, semi1),
    ]
    for ci in range(NCH):
        k = ci % 2
        cps[k].wait()
        buf = bufs[k]

        @plsc.parallel_loop(0, CR, step=1, unroll=2)
        def _h(r):
            for u in range(8):
                v = buf[r, pl.ds(u * 16, 16)]
                plsc.addupdate_scatter(hist, [v * 16 + lane], ones)

        if ci + 2 < NCH:
            cps[k] = pltpu.async_copy(
                mask_hbm.at[pl.ds(row0(ci + 2), CR), :], bufs[k], semis[k])

    cps = [
        pltpu.async_copy(mask_hbm.at[pl.ds(row0(0), CR), :], buf0, semi0),
        pltpu.async_copy(mask_hbm.at[pl.ds(row0(1), CR), :], buf1, semi1),
    ]

    pltpu.sync_copy(hist, shist.at[pl.ds(s * HWORDS, HWORDS)])
    plsc.subcore_barrier()
    pltpu.sync_copy(shist.at[pl.ds((s ^ 1) * HWORDS, HWORDS)], hist2)

    cls0 = lane * 16
    cls1 = (lane + 16) * 16
    cnt0 = zeros
    cnt1 = zeros
    for l in range(16):
        cnt0 = cnt0 + plsc.load_gather(hist, [cls0 + l])
        cnt0 = cnt0 + plsc.load_gather(hist2, [cls0 + l])
        cnt1 = cnt1 + plsc.load_gather(hist, [cls1 + l])
        cnt1 = cnt1 + plsc.load_gather(hist2, [cls1 + l])

    inv0 = jnp.where(cnt0 > 0.0, ones / jnp.maximum(cnt0, 1.0), zeros)
    inv1 = jnp.where(cnt1 > 0.0, ones / jnp.maximum(cnt1, 1.0), zeros)
    denom = lax.broadcast(jnp.sum(inv0 + inv1), (16,))
    wtab[pl.ds(0, 16)] = inv0 / denom
    wtab[pl.ds(16, 16)] = inv1 / denom

    wrs = [None, None]
    for ci in range(NCH):
        k = ci % 2
        cps[k].wait()
        if wrs[k] is not None:
            wrs[k].wait()
        buf = bufs[k]
        obuf = obufs[k]

        @plsc.parallel_loop(0, CR, step=1, unroll=2)
        def _g(r):
            for u in range(8):
                v = buf[r, pl.ds(u * 16, 16)]
                obuf[r, pl.ds(u * 16, 16)] = plsc.load_gather(wtab, [v])

        wrs[k] = pltpu.async_copy(
            obuf, out_hbm.at[pl.ds(row0(ci), CR), :], semos[k])
        if ci + 2 < NCH:
            cps[k] = pltpu.async_copy(
                mask_hbm.at[pl.ds(row0(ci + 2), CR), :], bufs[k], semis[k])

    wrs[0].wait()
    wrs[1].wait()


def kernel(inputs):
    mask = inputs.astype(jnp.int32).reshape(ROWS, 128)
    out = _sc_weight_map(mask)
    return out.reshape(NB, 512, 512, 1)
